# baseline (speedup 1.0000x reference)
import jax
import jax.numpy as jnp
from jax import lax
from jax.experimental import pallas as pl
from jax.experimental.pallas import tpu as pltpu
from jax.experimental.pallas import tpu_sc as plsc

_N = 10000
_E = 320000
_D = 128
_NB = 15
_INV_DDOF = 1.0 / (_D - 1)

_NC = 2
_NS = 16
_NW = _NC * _NS
_EW = _E // _NW
_CH = 40
_NCHUNK = _EW // _CH
_FB = 624
_TAIL = _N - _NS * _FB

_LANES = 16


def _rms(h):
    m = jnp.mean(h, axis=-1, keepdims=True)
    c = h - m
    v = jnp.sum(c * c, axis=-1, keepdims=True) * _INV_DDOF
    return h / jnp.sqrt(v)


def _edge_body(x2_hbm, ea_hbm, src_hbm, dst_hbm, out_hbm,
               accum, slab_s, slab_d, rows0, rows1, ea0, ea1,
               sem_g, sem_e, sem_s):
    cid = lax.axis_index("c")
    sid = lax.axis_index("s")
    wid = sid * _NC + cid
    zero = jnp.zeros((_LANES,), jnp.float32)

    pltpu.sync_copy(src_hbm.at[pl.ds(wid * _EW, _EW)], slab_s)
    pltpu.sync_copy(dst_hbm.at[pl.ds(wid * _EW, _EW)], slab_d)

    def _zero_row(e, car):
        for j in range(_D // _LANES):
            rows0[e, pl.ds(j * _LANES, _LANES)] = zero
        return car
    lax.fori_loop(0, _CH, _zero_row, 0)
    fbase = sid * _FB
    for r in range(_FB // _CH):
        pltpu.sync_copy(rows0, accum.at[pl.ds(fbase + r * _CH, _CH)])
    rem = _FB % _CH
    if rem:
        pltpu.sync_copy(rows0.at[pl.ds(0, rem)],
                        accum.at[pl.ds(fbase + _FB - rem, rem)])

    @pl.when(sid == _NS - 1)
    def _zero_tail():
        pltpu.sync_copy(rows0.at[pl.ds(0, _TAIL)],
                        accum.at[pl.ds(_NS * _FB, _TAIL)])
    plsc.subcore_barrier()

    def _fire(g, rows_b, ea_b):
        pltpu.async_copy(x2_hbm.at[slab_s.at[pl.ds(g * _CH, _CH)]],
                         rows_b, sem_g)
        pltpu.async_copy(ea_hbm.at[pl.ds(wid * _EW + g * _CH, _CH)],
                         ea_b, sem_e)

    def _drain(sem, buf):
        pltpu.make_async_copy(ea_hbm.at[pl.ds(0, _CH)], buf, sem).wait()

    def _fuse(rows_b, ea_b):
        def _f(eq, car):
            for u in range(4):
                e = eq * 4 + u
                for j in range(_D // _LANES):
                    s = pl.ds(j * _LANES, _LANES)
                    rows_b[e, s] = jnp.maximum(rows_b[e, s] + ea_b[e, s], 0.0)
            return car
        lax.fori_loop(0, _CH // 4, _f, 0)

    _fire(0, rows0, ea0)
    _fire(1, rows1, ea1)

    def _pair(p, car):
        g0 = 2 * p
        g1 = g0 + 1
        _drain(sem_g, rows0)
        _drain(sem_e, ea0)
        _fuse(rows0, ea0)
        sc0 = pltpu.async_copy(
            rows0, accum.at[slab_d.at[pl.ds(g0 * _CH, _CH)]], sem_s,
            add=True)
        _drain(sem_g, rows1)
        _drain(sem_e, ea1)
        _fuse(rows1, ea1)
        sc1 = pltpu.async_copy(
            rows1, accum.at[slab_d.at[pl.ds(g1 * _CH, _CH)]], sem_s,
            add=True)

        @pl.when(p < _NCHUNK // 2 - 1)
        def _refill():
            sc0.wait()
            _fire(g0 + 2, rows0, ea0)
            sc1.wait()
            _fire(g1 + 2, rows1, ea1)
        return car
    lax.fori_loop(0, _NCHUNK // 2, _pair, 0)
    _drain(sem_s, rows0)
    _drain(sem_s, rows1)
    plsc.subcore_barrier()

    for r in range(_FB // _CH):
        pltpu.sync_copy(accum.at[pl.ds(fbase + r * _CH, _CH)], rows0)
        pltpu.sync_copy(rows0, out_hbm.at[cid, pl.ds(fbase + r * _CH, _CH)])
    if rem:
        pltpu.sync_copy(accum.at[pl.ds(fbase + _FB - rem, rem)],
                        rows0.at[pl.ds(0, rem)])
        pltpu.sync_copy(rows0.at[pl.ds(0, rem)],
                        out_hbm.at[cid, pl.ds(fbase + _FB - rem, rem)])

    @pl.when(sid == _NS - 1)
    def _flush_tail():
        pltpu.sync_copy(accum.at[pl.ds(_NS * _FB, _TAIL)],
                        rows0.at[pl.ds(0, _TAIL)])
        pltpu.sync_copy(rows0.at[pl.ds(0, _TAIL)],
                        out_hbm.at[cid, pl.ds(_NS * _FB, _TAIL)])


def _edge_pass(x2, ea, src, dst):
    mesh = plsc.VectorSubcoreMesh(core_axis_name="c", subcore_axis_name="s")
    f = pl.kernel(
        _edge_body,
        out_type=jax.ShapeDtypeStruct((_NC, _N, _D), jnp.float32),
        mesh=mesh,
        scratch_types=[
            pltpu.VMEM_SHARED((_N, _D), jnp.float32),
            pltpu.VMEM((_EW,), jnp.int32),
            pltpu.VMEM((_EW,), jnp.int32),
            pltpu.VMEM((_CH, _D), jnp.float32),
            pltpu.VMEM((_CH, _D), jnp.float32),
            pltpu.VMEM((_CH, _D), jnp.float32),
            pltpu.VMEM((_CH, _D), jnp.float32),
            pltpu.SemaphoreType.DMA,
            pltpu.SemaphoreType.DMA,
            pltpu.SemaphoreType.DMA,
        ],
    )
    return f(x2, ea, src, dst)


_RT_TC = 1000
_ET_TC = 4000


def _embed_call(x, W, b):
    def body(x_ref, w_ref, b_ref, h_ref, x2_ref):
        xl = jnp.log(x_ref[...] + 1.0)
        h = jnp.dot(xl.astype(jnp.bfloat16), w_ref[...].astype(jnp.bfloat16),
                    preferred_element_type=jnp.float32)
        h = h + b_ref[...]
        h_ref[...] = h
        x2_ref[...] = _rms(h)
    return pl.pallas_call(
        body,
        grid=(_N // _RT_TC,),
        in_specs=[
            pl.BlockSpec((_RT_TC, _D), lambda t: (t, 0)),
            pl.BlockSpec((_D, _D), lambda t: (0, 0)),
            pl.BlockSpec((1, _D), lambda t: (0, 0)),
        ],
        out_specs=[
            pl.BlockSpec((_RT_TC, _D), lambda t: (t, 0)),
            pl.BlockSpec((_RT_TC, _D), lambda t: (t, 0)),
        ],
        out_shape=[
            jax.ShapeDtypeStruct((_N, _D), jnp.float32),
            jax.ShapeDtypeStruct((_N, _D), jnp.float32),
        ],
    )(x, W, b)


def _ea_call(edge_attr, Wi, bi):
    def body(a_ref, w_ref, b_ref, o_ref):
        al = jnp.log(a_ref[...] + 1.0)
        o_ref[...] = jnp.dot(al.astype(jnp.bfloat16), w_ref[...].astype(jnp.bfloat16),
                             preferred_element_type=jnp.float32) + b_ref[...]
    return pl.pallas_call(
        body,
        grid=(_E // _ET_TC,),
        in_specs=[
            pl.BlockSpec((_ET_TC, 4), lambda t: (t, 0)),
            pl.BlockSpec((4, _D), lambda t: (0, 0)),
            pl.BlockSpec((1, _D), lambda t: (0, 0)),
        ],
        out_specs=pl.BlockSpec((_ET_TC, _D), lambda t: (t, 0)),
        out_shape=jax.ShapeDtypeStruct((_E, _D), jnp.float32),
    )(edge_attr, Wi, bi)


def _mlp_call(x2, parts, h, W1, b1, W2, b2, W3, b3):
    def body(x2_ref, p_ref, h_ref, w1, b1_, w2, b2_, w3, b3_,
             hn_ref, x2n_ref):
        y = x2_ref[...] + p_ref[0] + p_ref[1]
        y = jnp.maximum(jnp.dot(y.astype(jnp.bfloat16), w1[...].astype(jnp.bfloat16),
                                preferred_element_type=jnp.float32) + b1_[...], 0.0)
        y = jnp.maximum(jnp.dot(y.astype(jnp.bfloat16), w2[...].astype(jnp.bfloat16),
                                preferred_element_type=jnp.float32) + b2_[...], 0.0)
        y = jnp.dot(y.astype(jnp.bfloat16), w3[...].astype(jnp.bfloat16),
                    preferred_element_type=jnp.float32) + b3_[...]
        y = _rms(y)
        hn = h_ref[...] + y
        hn_ref[...] = hn
        x2n_ref[...] = _rms(hn)
    wspec = pl.BlockSpec((_D, _D), lambda t: (0, 0))
    bspec = pl.BlockSpec((1, _D), lambda t: (0, 0))
    nspec = pl.BlockSpec((_RT_TC, _D), lambda t: (t, 0))
    pspec = pl.BlockSpec((_NC, _RT_TC, _D), lambda t: (0, t, 0))
    return pl.pallas_call(
        body,
        grid=(_N // _RT_TC,),
        in_specs=[nspec, pspec, nspec, wspec, bspec, wspec, bspec, wspec,
                  bspec],
        out_specs=[nspec, nspec],
        out_shape=[
            jax.ShapeDtypeStruct((_N, _D), jnp.float32),
            jax.ShapeDtypeStruct((_N, _D), jnp.float32),
        ],
    )(x2, parts, h, W1, b1, W2, b2, W3, b3)


def _out_call(h, out_W, out_b, scale):
    def body(h_ref, w_ref, b_ref, s_ref, o_ref):
        hs = h_ref[...] / jnp.sqrt(jnp.float32(_NB))
        t = jnp.dot(hs.astype(jnp.bfloat16), w_ref[...].astype(jnp.bfloat16),
                    preferred_element_type=jnp.float32) + b_ref[...]
        o_ref[...] = jnp.sum(t * s_ref[...], axis=-1, keepdims=True)
    return pl.pallas_call(
        body,
        grid=(_N // _RT_TC,),
        in_specs=[
            pl.BlockSpec((_RT_TC, _D), lambda t: (t, 0)),
            pl.BlockSpec((_D, 10), lambda t: (0, 0)),
            pl.BlockSpec((1, 10), lambda t: (0, 0)),
            pl.BlockSpec((1, 10), lambda t: (0, 0)),
        ],
        out_specs=pl.BlockSpec((_RT_TC, 1), lambda t: (t, 0)),
        out_shape=jax.ShapeDtypeStruct((_N, 1), jnp.float32),
    )(h, out_W, out_b, scale)


def kernel(x, edge_index, edge_attr, embed_W, embed_b, edge_W, edge_b,
           mlp_W1, mlp_b1, mlp_W2, mlp_b2, mlp_W3, mlp_b3, out_W, out_b,
           scale):
    src = edge_index[0]
    dst = edge_index[1]
    h, x2 = _embed_call(x, embed_W, embed_b.reshape(1, _D))
    for i in range(_NB):
        ea = _ea_call(edge_attr, edge_W[i], edge_b[i].reshape(1, _D))
        parts = _edge_pass(x2, ea, src, dst)
        h, x2 = _mlp_call(x2, parts, h,
                          mlp_W1[i], mlp_b1[i].reshape(1, _D),
                          mlp_W2[i], mlp_b2[i].reshape(1, _D),
                          mlp_W3[i], mlp_b3[i].reshape(1, _D))
    return _out_call(h, out_W, out_b.reshape(1, 10), scale.reshape(1, 10))


# speedup vs baseline: 1.0008x; 1.0008x over previous
import jax
import jax.numpy as jnp
from jax import lax
from jax.experimental import pallas as pl
from jax.experimental.pallas import tpu as pltpu
from jax.experimental.pallas import tpu_sc as plsc

_N = 10000
_E = 320000
_D = 128
_NB = 15
_INV_DDOF = 1.0 / (_D - 1)

_NC = 2
_NS = 16
_NW = _NC * _NS
_EW = _E // _NW
_CH = 40
_NCHUNK = _EW // _CH
_FB = 624
_TAIL = _N - _NS * _FB

_LANES = 16


def _rms(h):
    m = jnp.mean(h, axis=-1, keepdims=True)
    c = h - m
    v = jnp.sum(c * c, axis=-1, keepdims=True) * _INV_DDOF
    return h / jnp.sqrt(v)


def _edge_body(x2_hbm, ea_hbm, src_hbm, dst_hbm, out_hbm,
               accum, slab_s, slab_d, rows0, rows1, ea0, ea1,
               sem_g, sem_e, sem_s):
    cid = lax.axis_index("c")
    sid = lax.axis_index("s")
    wid = sid * _NC + cid
    zero = jnp.zeros((_LANES,), jnp.float32)

    pltpu.sync_copy(src_hbm.at[pl.ds(wid * _EW, _EW)], slab_s)
    pltpu.sync_copy(dst_hbm.at[pl.ds(wid * _EW, _EW)], slab_d)

    def _zero_row(e, car):
        for j in range(_D // _LANES):
            rows0[e, pl.ds(j * _LANES, _LANES)] = zero
        return car
    lax.fori_loop(0, _CH, _zero_row, 0)
    fbase = sid * _FB
    for r in range(_FB // _CH):
        pltpu.sync_copy(rows0, accum.at[pl.ds(fbase + r * _CH, _CH)])
    rem = _FB % _CH
    if rem:
        pltpu.sync_copy(rows0.at[pl.ds(0, rem)],
                        accum.at[pl.ds(fbase + _FB - rem, rem)])

    @pl.when(sid == _NS - 1)
    def _zero_tail():
        pltpu.sync_copy(rows0.at[pl.ds(0, _TAIL)],
                        accum.at[pl.ds(_NS * _FB, _TAIL)])
    plsc.subcore_barrier()

    def _fire(g, rows_b, ea_b):
        pltpu.async_copy(x2_hbm.at[slab_s.at[pl.ds(g * _CH, _CH)]],
                         rows_b, sem_g)
        pltpu.async_copy(ea_hbm.at[pl.ds(wid * _EW + g * _CH, _CH)],
                         ea_b, sem_e)

    def _drain(sem, buf):
        pltpu.make_async_copy(ea_hbm.at[pl.ds(0, _CH)], buf, sem).wait()

    def _fuse(rows_b, ea_b):
        def _f(e, car):
            for j in range(_D // _LANES):
                s = pl.ds(j * _LANES, _LANES)
                rows_b[e, s] = jnp.maximum(rows_b[e, s] + ea_b[e, s], 0.0)
            return car
        lax.fori_loop(0, _CH, _f, 0)

    _fire(0, rows0, ea0)
    _fire(1, rows1, ea1)

    def _pair(p, car):
        g0 = 2 * p
        g1 = g0 + 1
        _drain(sem_g, rows0)
        _drain(sem_e, ea0)
        _fuse(rows0, ea0)
        sc0 = pltpu.async_copy(
            rows0, accum.at[slab_d.at[pl.ds(g0 * _CH, _CH)]], sem_s,
            add=True)
        _drain(sem_g, rows1)
        _drain(sem_e, ea1)
        _fuse(rows1, ea1)
        sc1 = pltpu.async_copy(
            rows1, accum.at[slab_d.at[pl.ds(g1 * _CH, _CH)]], sem_s,
            add=True)

        @pl.when(p < _NCHUNK // 2 - 1)
        def _refill():
            sc0.wait()
            _fire(g0 + 2, rows0, ea0)
            sc1.wait()
            _fire(g1 + 2, rows1, ea1)
        return car
    lax.fori_loop(0, _NCHUNK // 2, _pair, 0)
    _drain(sem_s, rows0)
    _drain(sem_s, rows1)
    plsc.subcore_barrier()

    for r in range(_FB // _CH):
        pltpu.sync_copy(accum.at[pl.ds(fbase + r * _CH, _CH)], rows0)
        pltpu.sync_copy(rows0, out_hbm.at[cid, pl.ds(fbase + r * _CH, _CH)])
    if rem:
        pltpu.sync_copy(accum.at[pl.ds(fbase + _FB - rem, rem)],
                        rows0.at[pl.ds(0, rem)])
        pltpu.sync_copy(rows0.at[pl.ds(0, rem)],
                        out_hbm.at[cid, pl.ds(fbase + _FB - rem, rem)])

    @pl.when(sid == _NS - 1)
    def _flush_tail():
        pltpu.sync_copy(accum.at[pl.ds(_NS * _FB, _TAIL)],
                        rows0.at[pl.ds(0, _TAIL)])
        pltpu.sync_copy(rows0.at[pl.ds(0, _TAIL)],
                        out_hbm.at[cid, pl.ds(_NS * _FB, _TAIL)])


def _edge_pass(x2, ea, src, dst):
    mesh = plsc.VectorSubcoreMesh(core_axis_name="c", subcore_axis_name="s")
    f = pl.kernel(
        _edge_body,
        out_type=jax.ShapeDtypeStruct((_NC, _N, _D), jnp.float32),
        mesh=mesh,
        scratch_types=[
            pltpu.VMEM_SHARED((_N, _D), jnp.float32),
            pltpu.VMEM((_EW,), jnp.int32),
            pltpu.VMEM((_EW,), jnp.int32),
            pltpu.VMEM((_CH, _D), jnp.float32),
            pltpu.VMEM((_CH, _D), jnp.float32),
            pltpu.VMEM((_CH, _D), jnp.float32),
            pltpu.VMEM((_CH, _D), jnp.float32),
            pltpu.SemaphoreType.DMA,
            pltpu.SemaphoreType.DMA,
            pltpu.SemaphoreType.DMA,
        ],
    )
    return f(x2, ea, src, dst)


_RT_TC = 1000
_ET_TC = 4000


def _embed_call(x, W, b):
    def body(x_ref, w_ref, b_ref, h_ref, x2_ref):
        xl = jnp.log(x_ref[...] + 1.0)
        h = jnp.dot(xl.astype(jnp.bfloat16), w_ref[...].astype(jnp.bfloat16),
                    preferred_element_type=jnp.float32)
        h = h + b_ref[...]
        h_ref[...] = h
        x2_ref[...] = _rms(h)
    return pl.pallas_call(
        body,
        grid=(_N // _RT_TC,),
        in_specs=[
            pl.BlockSpec((_RT_TC, _D), lambda t: (t, 0)),
            pl.BlockSpec((_D, _D), lambda t: (0, 0)),
            pl.BlockSpec((1, _D), lambda t: (0, 0)),
        ],
        out_specs=[
            pl.BlockSpec((_RT_TC, _D), lambda t: (t, 0)),
            pl.BlockSpec((_RT_TC, _D), lambda t: (t, 0)),
        ],
        out_shape=[
            jax.ShapeDtypeStruct((_N, _D), jnp.float32),
            jax.ShapeDtypeStruct((_N, _D), jnp.float32),
        ],
    )(x, W, b)


def _ea_call(edge_attr, Wi, bi):
    def body(a_ref, w_ref, b_ref, o_ref):
        al = jnp.log(a_ref[...] + 1.0)
        o_ref[...] = jnp.dot(al.astype(jnp.bfloat16), w_ref[...].astype(jnp.bfloat16),
                             preferred_element_type=jnp.float32) + b_ref[...]
    return pl.pallas_call(
        body,
        grid=(_E // _ET_TC,),
        in_specs=[
            pl.BlockSpec((_ET_TC, 4), lambda t: (t, 0)),
            pl.BlockSpec((4, _D), lambda t: (0, 0)),
            pl.BlockSpec((1, _D), lambda t: (0, 0)),
        ],
        out_specs=pl.BlockSpec((_ET_TC, _D), lambda t: (t, 0)),
        out_shape=jax.ShapeDtypeStruct((_E, _D), jnp.float32),
    )(edge_attr, Wi, bi)


def _mlp_call(x2, parts, h, W1, b1, W2, b2, W3, b3):
    def body(x2_ref, p_ref, h_ref, w1, b1_, w2, b2_, w3, b3_,
             hn_ref, x2n_ref):
        y = x2_ref[...] + p_ref[0] + p_ref[1]
        y = jnp.maximum(jnp.dot(y.astype(jnp.bfloat16), w1[...].astype(jnp.bfloat16),
                                preferred_element_type=jnp.float32) + b1_[...], 0.0)
        y = jnp.maximum(jnp.dot(y.astype(jnp.bfloat16), w2[...].astype(jnp.bfloat16),
                                preferred_element_type=jnp.float32) + b2_[...], 0.0)
        y = jnp.dot(y.astype(jnp.bfloat16), w3[...].astype(jnp.bfloat16),
                    preferred_element_type=jnp.float32) + b3_[...]
        y = _rms(y)
        hn = h_ref[...] + y
        hn_ref[...] = hn
        x2n_ref[...] = _rms(hn)
    wspec = pl.BlockSpec((_D, _D), lambda t: (0, 0))
    bspec = pl.BlockSpec((1, _D), lambda t: (0, 0))
    nspec = pl.BlockSpec((_RT_TC, _D), lambda t: (t, 0))
    pspec = pl.BlockSpec((_NC, _RT_TC, _D), lambda t: (0, t, 0))
    return pl.pallas_call(
        body,
        grid=(_N // _RT_TC,),
        in_specs=[nspec, pspec, nspec, wspec, bspec, wspec, bspec, wspec,
                  bspec],
        out_specs=[nspec, nspec],
        out_shape=[
            jax.ShapeDtypeStruct((_N, _D), jnp.float32),
            jax.ShapeDtypeStruct((_N, _D), jnp.float32),
        ],
    )(x2, parts, h, W1, b1, W2, b2, W3, b3)


def _out_call(h, out_W, out_b, scale):
    def body(h_ref, w_ref, b_ref, s_ref, o_ref):
        hs = h_ref[...] / jnp.sqrt(jnp.float32(_NB))
        t = jnp.dot(hs.astype(jnp.bfloat16), w_ref[...].astype(jnp.bfloat16),
                    preferred_element_type=jnp.float32) + b_ref[...]
        o_ref[...] = jnp.sum(t * s_ref[...], axis=-1, keepdims=True)
    return pl.pallas_call(
        body,
        grid=(_N // _RT_TC,),
        in_specs=[
            pl.BlockSpec((_RT_TC, _D), lambda t: (t, 0)),
            pl.BlockSpec((_D, 10), lambda t: (0, 0)),
            pl.BlockSpec((1, 10), lambda t: (0, 0)),
            pl.BlockSpec((1, 10), lambda t: (0, 0)),
        ],
        out_specs=pl.BlockSpec((_RT_TC, 1), lambda t: (t, 0)),
        out_shape=jax.ShapeDtypeStruct((_N, 1), jnp.float32),
    )(h, out_W, out_b, scale)


def kernel(x, edge_index, edge_attr, embed_W, embed_b, edge_W, edge_b,
           mlp_W1, mlp_b1, mlp_W2, mlp_b2, mlp_W3, mlp_b3, out_W, out_b,
           scale):
    src = edge_index[0]
    dst = edge_index[1]
    h, x2 = _embed_call(x, embed_W, embed_b.reshape(1, _D))
    for i in range(_NB):
        ea = _ea_call(edge_attr, edge_W[i], edge_b[i].reshape(1, _D))
        parts = _edge_pass(x2, ea, src, dst)
        h, x2 = _mlp_call(x2, parts, h,
                          mlp_W1[i], mlp_b1[i].reshape(1, _D),
                          mlp_W2[i], mlp_b2[i].reshape(1, _D),
                          mlp_W3[i], mlp_b3[i].reshape(1, _D))
    return _out_call(h, out_W, out_b.reshape(1, 10), scale.reshape(1, 10))
